# baseline (speedup 1.0000x reference)
import functools

import jax
import jax.numpy as jnp
from jax import lax
from jax.experimental import pallas as pl
from jax.experimental.pallas import tpu as pltpu
from jax.experimental.pallas import tpu_sc as plsc

_V = 100000
_B = 16384
_L = 200

_NC = 2
_NS = 16
_NW = _NC * _NS
_GROUPS = _B // 16
_GPW = _GROUPS // _NW
_GPC = 2
_NCHUNK = _GPW // _GPC
_CHW = _GPC * _L * 16


def _diff_body(p_ref, n_ref, o_ref):
    o_ref[...] = p_ref[...] - n_ref[...]


_diff_call = pl.pallas_call(
    _diff_body,
    out_shape=jax.ShapeDtypeStruct((_V,), jnp.float32),
)


_UNROLL = 20


def _sc_body(d_hbm, idx_hbm, out_hbm, d_vmem, idx_a, idx_b, out_vmem,
             sem_d, sem_a, sem_b):
    wid = lax.axis_index("c") * _NS + lax.axis_index("s")

    d_copy = pltpu.async_copy(d_hbm, d_vmem, sem_d)
    bufs = (idx_a, idx_b)
    sems = (sem_a, sem_b)
    copies = [None, None]
    rows_per_chunk = _GPC * 16
    base_row = wid * _GPW * 16
    copies[0] = pltpu.async_copy(
        idx_hbm.at[pl.ds(base_row, rows_per_chunk)], idx_a, sem_a)
    d_copy.wait()

    lane = lax.iota(jnp.int32, 16)

    for c in range(_NCHUNK):
        cur = c % 2
        if c + 1 < _NCHUNK:
            nxt = (c + 1) % 2
            copies[nxt] = pltpu.async_copy(
                idx_hbm.at[pl.ds(base_row + (c + 1) * rows_per_chunk,
                                 rows_per_chunk)],
                bufs[nxt], sems[nxt])
        copies[cur].wait()
        ibuf = bufs[cur]
        for g in range(_GPC):
            rowv = lane + (g * 16)

            def body(_, carry, _rowv=rowv, _ibuf=ibuf):
                rel0, a0, a1, a2, a3 = carry
                accs = [a0, a1, a2, a3]
                for u in range(_UNROLL):
                    relu = rel0 + u
                    relu = jnp.where(relu >= _L, relu - _L, relu)
                    idxv = plsc.load_gather(_ibuf, [_rowv, relu])
                    vals = plsc.load_gather(d_vmem, [idxv])
                    accs[u % 4] = accs[u % 4] + vals
                rel0 = rel0 + _UNROLL
                rel0 = jnp.where(rel0 >= _L, rel0 - _L, rel0)
                return (rel0, *accs)

            zero = jnp.zeros((16,), jnp.float32)
            _, a0, a1, a2, a3 = lax.fori_loop(
                0, _L // _UNROLL, body, (lane, zero, zero, zero, zero))
            out_vmem[pl.ds((c * _GPC + g) * 16, 16)] = (a0 + a1) + (a2 + a3)

    pltpu.sync_copy(out_vmem, out_hbm.at[pl.ds(wid * _GPW * 16, _GPW * 16)])


_sc_call = pl.kernel(
    _sc_body,
    out_type=jax.ShapeDtypeStruct((_B,), jnp.float32),
    mesh=plsc.VectorSubcoreMesh(core_axis_name="c", subcore_axis_name="s"),
    compiler_params=pltpu.CompilerParams(needs_layout_passes=False,
                                         use_tc_tiling_on_sc=True),
    scratch_types=[
        pltpu.VMEM((_V,), jnp.float32),
        pltpu.VMEM((_GPC * 16, _L), jnp.int32),
        pltpu.VMEM((_GPC * 16, _L), jnp.int32),
        pltpu.VMEM((_GPW * 16,), jnp.float32),
        pltpu.SemaphoreType.DMA,
        pltpu.SemaphoreType.DMA,
        pltpu.SemaphoreType.DMA,
    ],
)


def kernel(indices, W_pos, W_neg):
    d = _diff_call(W_pos.reshape(_V), W_neg.reshape(_V))
    return _sc_call(d, indices.astype(jnp.int32))
